# baseline (speedup 1.0000x reference)
import jax
import jax.numpy as jnp
from jax.experimental import pallas as pl
from jax.experimental.pallas import tpu as pltpu

S = 128
SCALE = float(S // 2 - 2)


def _splat_kernel(x_ref, o_ref, cr_ref, cc_ref, w_ref):
    R = x_ref.shape[2]

    for b in range(2):
        pc = x_ref[b] * SCALE
        clf = jnp.trunc(pc)
        cli = clf.astype(jnp.int32)
        feat = 2.0 - jnp.abs(clf - pc).sum(axis=0)
        c0 = cli[0] - jnp.min(cli[0])
        c1 = cli[1] - jnp.min(cli[1])
        oob = (c0 >= S) | (c1 >= S)
        c0 = jnp.where(c0 >= S, 0, c0)
        c1 = jnp.where(c1 >= S, 0, c1)
        w = jnp.where(oob, 0.0, feat)
        cr_ref[:, b * S:(b + 1) * S] = c0.astype(jnp.int16)
        cc_ref[:, b * S:(b + 1) * S] = c1 + b * S
        w_ref[:, b * S:(b + 1) * S] = w.astype(jnp.bfloat16)

    iota_a = jax.lax.broadcasted_iota(jnp.int16, (S, 2 * S), 0)
    iota_ct = jax.lax.broadcasted_iota(jnp.int32, (2 * S, 2 * S), 1).astype(jnp.uint8)
    one8 = jnp.float8_e4m3fn(1.0)
    zero8 = jnp.float8_e4m3fn(0.0)
    zero = jnp.bfloat16(0.0)

    def body(i, acc):
        for h in range(32):
            base = i * 512 + h * 16
            tr = cr_ref[pl.ds(base, 16), :]
            tc = cc_ref[pl.ds(base, 16), :]
            tw = w_ref[pl.ds(base, 16), :]
            tct = jnp.transpose(tc).astype(jnp.uint8)
            for u in range(16):
                r_row = tr[u:u + 1, :]
                w_row = tw[u:u + 1, :]
                c_col = tct[:, u:u + 1]
                a_mat = jnp.where(r_row == iota_a, w_row, zero)
                ct_mat = jnp.where(c_col == iota_ct, one8, zero8)
                acc = acc + jax.lax.dot_general(
                    a_mat, ct_mat, (((1,), (0,)), ((), ())),
                    preferred_element_type=jnp.float32)
        return acc

    acc = jax.lax.fori_loop(0, R // 512, body,
                            jnp.zeros((S, 2 * S), jnp.float32))
    o_ref[0] = acc[:, :S]
    o_ref[1] = acc[:, S:]


def kernel(x):
    B, C, N = x.shape
    R = N // 128
    xr = jax.lax.slice(x, (0, 0, 0), (B, 2, N)).reshape(B, 2, R, 128)
    out = pl.pallas_call(
        _splat_kernel,
        grid=(B // 2,),
        in_specs=[pl.BlockSpec((2, 2, R, 128), lambda p: (p, 0, 0, 0))],
        out_specs=pl.BlockSpec((2, S, S), lambda p: (p, 0, 0)),
        out_shape=jax.ShapeDtypeStruct((B, S, S), jnp.float32),
        scratch_shapes=[
            pltpu.VMEM((R, 2 * S), jnp.int16),
            pltpu.VMEM((R, 2 * S), jnp.int32),
            pltpu.VMEM((R, 2 * S), jnp.bfloat16),
        ],
        compiler_params=pltpu.CompilerParams(
            dimension_semantics=("parallel",)),
    )(xr)
    return out[:, None, :, :]


# speedup vs baseline: 1.0114x; 1.0114x over previous
import jax
import jax.numpy as jnp
from jax.experimental import pallas as pl
from jax.experimental.pallas import tpu as pltpu

S = 128
SCALE = float(S // 2 - 2)


def _splat_kernel(x_ref, o_ref, cr_ref, cc_ref, w_ref):
    R = x_ref.shape[2]

    for b in range(2):
        pc = x_ref[b] * SCALE
        clf = jnp.trunc(pc)
        cli = clf.astype(jnp.int32)
        feat = 2.0 - jnp.abs(clf - pc).sum(axis=0)
        c0 = cli[0] - jnp.min(cli[0])
        c1 = cli[1] - jnp.min(cli[1])
        oob = (c0 >= S) | (c1 >= S)
        c0 = jnp.where(c0 >= S, 0, c0)
        c1 = jnp.where(c1 >= S, 0, c1)
        w = jnp.where(oob, 0.0, feat)
        cr_ref[:, b * S:(b + 1) * S] = c0.astype(jnp.int16)
        cc_ref[:, b * S:(b + 1) * S] = c1 + b * S
        w_ref[:, b * S:(b + 1) * S] = w.astype(jnp.bfloat16)

    iota_a = jax.lax.broadcasted_iota(jnp.int16, (S, 2 * S), 0)
    iota_ct = jax.lax.broadcasted_iota(jnp.int32, (2 * S, 2 * S), 1).astype(jnp.uint8)
    one8 = jnp.float8_e4m3fn(1.0)
    zero8 = jnp.float8_e4m3fn(0.0)
    zero = jnp.bfloat16(0.0)

    def body(i, acc):
        for h in range(16):
            base = i * 256 + h * 16
            tr = cr_ref[pl.ds(base, 16), :]
            tc = cc_ref[pl.ds(base, 16), :]
            tw = w_ref[pl.ds(base, 16), :]
            tct = jnp.transpose(tc).astype(jnp.uint8)
            for u in range(16):
                r_row = tr[u:u + 1, :]
                w_row = tw[u:u + 1, :]
                c_col = tct[:, u:u + 1]
                a_mat = jnp.where(r_row == iota_a, w_row, zero)
                ct_mat = jnp.where(c_col == iota_ct, one8, zero8)
                acc = acc + jax.lax.dot_general(
                    a_mat, ct_mat, (((1,), (0,)), ((), ())),
                    preferred_element_type=jnp.float32)
        return acc

    acc = jax.lax.fori_loop(0, R // 256, body,
                            jnp.zeros((S, 2 * S), jnp.float32))
    o_ref[0] = acc[:, :S]
    o_ref[1] = acc[:, S:]


def kernel(x):
    B, C, N = x.shape
    R = N // 128
    xr = jax.lax.slice(x, (0, 0, 0), (B, 2, N)).reshape(B, 2, R, 128)
    out = pl.pallas_call(
        _splat_kernel,
        grid=(B // 2,),
        in_specs=[pl.BlockSpec((2, 2, R, 128), lambda p: (p, 0, 0, 0))],
        out_specs=pl.BlockSpec((2, S, S), lambda p: (p, 0, 0)),
        out_shape=jax.ShapeDtypeStruct((B, S, S), jnp.float32),
        scratch_shapes=[
            pltpu.VMEM((R, 2 * S), jnp.int16),
            pltpu.VMEM((R, 2 * S), jnp.int32),
            pltpu.VMEM((R, 2 * S), jnp.bfloat16),
        ],
        compiler_params=pltpu.CompilerParams(
            dimension_semantics=("parallel",)),
    )(xr)
    return out[:, None, :, :]
